# baseline (speedup 1.0000x reference)
import functools

import jax
import jax.numpy as jnp
from jax import lax
from jax.experimental import pallas as pl
from jax.experimental.pallas import tpu as pltpu
from jax.experimental.pallas import tpu_sc as plsc

_N = 65536
_ALPHA = 50.0
_RGB_W = 1.0
_MASK_W = 2.0
_EIK_W = 0.1
_CSDF_W = 1.0
_CREG_W = 1.0

_NC = 2
_NS = 16
_NW = _NC * _NS
_L = 16
_R = _N // _NW
_R2 = _R // 2
_CH2 = _R2 // _L

_AX, _AY, _AZ, _BX, _BY, _BZ, _GX, _GY, _GZ, _NX, _NY, _NZ, \
    _PM, _GM, _SDF, _SH, _SD = range(17)


def _rsqrt(s):
    i = plsc.bitcast(s, jnp.int32)
    i = jnp.int32(0x5F3759DF) - (i >> 1)
    y = plsc.bitcast(i, jnp.float32)
    for _ in range(3):
        y = y * (1.5 - 0.5 * s * y * y)
    return y


def _softplus_neg(a):
    u = jnp.exp(-a)
    s = u / (2.0 + u)
    s2 = s * s
    return 2.0 * s * (1.0 + s2 * (1.0 / 3.0 + s2 * (
        1.0 / 5.0 + s2 * (1.0 / 7.0 + s2 * (1.0 / 9.0)))))


def _sc_body(y, z, out, xv, part_v, sem_a, sem_b):
    wid = lax.axis_index("s") * _NC + lax.axis_index("c")
    base = wid * _R

    half_a = [
        pltpu.async_copy(y.at[j, pl.ds(base, _R2)],
                         xv.at[pl.ds(j * _R, _R2)], sem_a)
        for j in range(12)
    ] + [
        pltpu.async_copy(z.at[pl.ds((j - 12) * _N + base, _R2)],
                         xv.at[pl.ds(j * _R, _R2)], sem_a)
        for j in range(12, 17)
    ]
    half_b = [
        pltpu.async_copy(y.at[j, pl.ds(base + _R2, _R2)],
                         xv.at[pl.ds(j * _R + _R2, _R2)], sem_b)
        for j in range(12)
    ] + [
        pltpu.async_copy(z.at[pl.ds((j - 12) * _N + base + _R2, _R2)],
                         xv.at[pl.ds(j * _R + _R2, _R2)], sem_b)
        for j in range(12, 17)
    ]

    zero = jnp.zeros((_L,), jnp.float32)

    def chunk(i, accs):
        a0, a1, a2, a3, a4, a5 = accs
        pmv = xv[pl.ds(_PM * _R + i * _L, _L)]
        gmv = xv[pl.ds(_GM * _R + i * _L, _L)]
        m = pmv * gmv

        d = (jnp.abs(xv[pl.ds(_AX * _R + i * _L, _L)] - xv[pl.ds(_BX * _R + i * _L, _L)]) +
             jnp.abs(xv[pl.ds(_AY * _R + i * _L, _L)] - xv[pl.ds(_BY * _R + i * _L, _L)]) +
             jnp.abs(xv[pl.ds(_AZ * _R + i * _L, _L)] - xv[pl.ds(_BZ * _R + i * _L, _L)]))
        a0 = a0 + d * m

        z = -_ALPHA * xv[pl.ds(_SDF * _R + i * _L, _L)]
        bce = jnp.maximum(z, 0.0) - z * gmv + _softplus_neg(jnp.abs(z))
        a1 = a1 + bce * (1.0 - m)

        gx = xv[pl.ds(_GX * _R + i * _L, _L)]
        gy = xv[pl.ds(_GY * _R + i * _L, _L)]
        gz = xv[pl.ds(_GZ * _R + i * _L, _L)]
        s = gx * gx + gy * gy + gz * gz
        ns = s * _rsqrt(jnp.maximum(s, 1e-30))
        t = ns - 1.0
        a2 = a2 + t * t

        shv = xv[pl.ds(_SH * _R + i * _L, _L)]
        sdv = xv[pl.ds(_SD * _R + i * _L, _L)]
        cm = jnp.where((shv < 0.0) & (sdv < 0.0), 1.0, 0.0)
        a3 = a3 + jnp.maximum(-shv, 0.0) * cm
        a4 = a4 + cm

        nx = xv[pl.ds(_NX * _R + i * _L, _L)]
        ny = xv[pl.ds(_NY * _R + i * _L, _L)]
        nz = xv[pl.ds(_NZ * _R + i * _L, _L)]
        a5 = a5 + (nx * nx + ny * ny + nz * nz) * (1.0 - cm)

        return (a0, a1, a2, a3, a4, a5)

    for c in half_a:
        c.wait()

    accs = plsc.parallel_loop(0, _CH2, unroll=4, carry=(zero,) * 6)(chunk)

    for c in half_b:
        c.wait()

    accs = plsc.parallel_loop(_CH2, 2 * _CH2, unroll=4, carry=accs)(chunk)

    for k in range(6):
        part_v[pl.ds(k * _L, _L)] = accs[k]
    outs = [
        pltpu.async_copy(part_v.at[pl.ds(k * _L, _L)],
                         out.at[pl.ds((k * _NW + wid) * _L, _L)], sem_a)
        for k in range(6)
    ]
    for c in outs:
        c.wait()


_sc_partials = functools.partial(
    pl.kernel,
    mesh=plsc.VectorSubcoreMesh(core_axis_name="c", subcore_axis_name="s"),
    out_type=jax.ShapeDtypeStruct((_NW * 6 * _L,), jnp.float32),
    compiler_params=pltpu.CompilerParams(
        needs_layout_passes=False,
        skip_device_barrier=True,
    ),
    scratch_types=[
        pltpu.VMEM((17 * _R,), jnp.float32),
        pltpu.VMEM((6 * _L,), jnp.float32),
        pltpu.SemaphoreType.DMA,
        pltpu.SemaphoreType.DMA,
    ],
)(_sc_body)


def _fin_body(x_ref, o_ref):
    p = [jnp.sum(x_ref[4 * k:4 * (k + 1), :]) for k in range(6)]
    n = float(_N)
    rgb_loss = p[0] / n
    mask_loss = (1.0 / _ALPHA) * p[1] / n
    eik_loss = p[2] / n
    contact_loss = p[3] / jnp.maximum(p[4], 1.0)
    contact_reg = p[5] / jnp.maximum((n - p[4]) * 3.0, 1.0)
    o_ref[0, 0] = (_RGB_W * rgb_loss + _MASK_W * mask_loss +
                   _EIK_W * eik_loss + _CSDF_W * contact_loss +
                   _CREG_W * contact_reg)


_finalize = pl.pallas_call(
    _fin_body,
    out_shape=jax.ShapeDtypeStruct((1, 1), jnp.float32),
    out_specs=pl.BlockSpec(memory_space=pltpu.SMEM),
)


@jax.jit
def kernel(rgb_values, rgb_gt, pred_mask, gt_mask, sdf_output, grad_theta,
           sdf_head, sdf_hand, nonrigid_deformation):
    y = jnp.concatenate([rgb_values.T, rgb_gt.T, grad_theta.T,
                         nonrigid_deformation.T], axis=0)
    z = jnp.concatenate([
        pred_mask.astype(jnp.float32),
        gt_mask.astype(jnp.float32),
        sdf_output.reshape(-1),
        sdf_head,
        sdf_hand,
    ])
    parts = _sc_partials(y, z)
    total = _finalize(parts.reshape(_NW * 6 * _L // 128, 128))
    return total[0, 0]


# speedup vs baseline: 1.0998x; 1.0998x over previous
import functools

import jax
import jax.numpy as jnp
from jax import lax
from jax.experimental import pallas as pl
from jax.experimental.pallas import tpu as pltpu
from jax.experimental.pallas import tpu_sc as plsc

_N = 65536
_ALPHA = 50.0
_RGB_W = 1.0
_MASK_W = 2.0
_EIK_W = 0.1
_CSDF_W = 1.0
_CREG_W = 1.0

_NC = 2
_NS = 16
_NW = _NC * _NS
_L = 16
_R = _N // _NW
_CH = _R // _L

_AX, _AY, _AZ, _BX, _BY, _BZ, _GX, _GY, _GZ, _NX, _NY, _NZ, \
    _PM, _GM, _SDF, _SH, _SD = range(17)


def _rsqrt(s):
    i = plsc.bitcast(s, jnp.int32)
    i = jnp.int32(0x5F3759DF) - (i >> 1)
    y = plsc.bitcast(i, jnp.float32)
    for _ in range(3):
        y = y * (1.5 - 0.5 * s * y * y)
    return y


def _softplus_neg(a):
    u = jnp.exp(-a)
    s = u / (2.0 + u)
    s2 = s * s
    return 2.0 * s * (1.0 + s2 * (1.0 / 3.0 + s2 * (
        1.0 / 5.0 + s2 * (1.0 / 7.0 + s2 * (1.0 / 9.0)))))


def _sc_body(y, pm, gm, sdf, sh, sd, out, xv, part_v, sem_a):
    wid = lax.axis_index("s") * _NC + lax.axis_index("c")
    base = wid * _R

    copies = [
        pltpu.async_copy(y.at[j, pl.ds(base, _R)],
                         xv.at[pl.ds(j * _R, _R)], sem_a)
        for j in range(12)
    ] + [
        pltpu.async_copy(arr.at[pl.ds(base, _R)],
                         xv.at[pl.ds(j * _R, _R)], sem_a)
        for j, arr in ((_PM, pm), (_GM, gm), (_SDF, sdf), (_SH, sh), (_SD, sd))
    ]
    for c in copies:
        c.wait()

    zero = jnp.zeros((_L,), jnp.float32)

    def chunk(i, accs):
        a0, a1, a2, a3, a4, a5 = accs
        pmv = xv[pl.ds(_PM * _R + i * _L, _L)]
        gmv = xv[pl.ds(_GM * _R + i * _L, _L)]
        m = pmv * gmv

        d = (jnp.abs(xv[pl.ds(_AX * _R + i * _L, _L)] - xv[pl.ds(_BX * _R + i * _L, _L)]) +
             jnp.abs(xv[pl.ds(_AY * _R + i * _L, _L)] - xv[pl.ds(_BY * _R + i * _L, _L)]) +
             jnp.abs(xv[pl.ds(_AZ * _R + i * _L, _L)] - xv[pl.ds(_BZ * _R + i * _L, _L)]))
        a0 = a0 + d * m

        z = -_ALPHA * xv[pl.ds(_SDF * _R + i * _L, _L)]
        bce = jnp.maximum(z, 0.0) - z * gmv + _softplus_neg(jnp.abs(z))
        a1 = a1 + bce * (1.0 - m)

        gx = xv[pl.ds(_GX * _R + i * _L, _L)]
        gy = xv[pl.ds(_GY * _R + i * _L, _L)]
        gz = xv[pl.ds(_GZ * _R + i * _L, _L)]
        s = gx * gx + gy * gy + gz * gz
        ns = s * _rsqrt(jnp.maximum(s, 1e-30))
        t = ns - 1.0
        a2 = a2 + t * t

        shv = xv[pl.ds(_SH * _R + i * _L, _L)]
        sdv = xv[pl.ds(_SD * _R + i * _L, _L)]
        cm = jnp.where((shv < 0.0) & (sdv < 0.0), 1.0, 0.0)
        a3 = a3 + jnp.maximum(-shv, 0.0) * cm
        a4 = a4 + cm

        nx = xv[pl.ds(_NX * _R + i * _L, _L)]
        ny = xv[pl.ds(_NY * _R + i * _L, _L)]
        nz = xv[pl.ds(_NZ * _R + i * _L, _L)]
        a5 = a5 + (nx * nx + ny * ny + nz * nz) * (1.0 - cm)

        return (a0, a1, a2, a3, a4, a5)

    accs = lax.fori_loop(0, _CH, chunk, (zero,) * 6)

    for k in range(6):
        part_v[pl.ds(k * _L, _L)] = accs[k]
    outs = [
        pltpu.async_copy(part_v.at[pl.ds(k * _L, _L)],
                         out.at[pl.ds((k * _NW + wid) * _L, _L)], sem_a)
        for k in range(6)
    ]
    for c in outs:
        c.wait()


_sc_partials = functools.partial(
    pl.kernel,
    mesh=plsc.VectorSubcoreMesh(core_axis_name="c", subcore_axis_name="s"),
    out_type=jax.ShapeDtypeStruct((_NW * 6 * _L,), jnp.float32),
    compiler_params=pltpu.CompilerParams(
        needs_layout_passes=False,
        skip_device_barrier=True,
    ),
    scratch_types=[
        pltpu.VMEM((17 * _R,), jnp.float32),
        pltpu.VMEM((6 * _L,), jnp.float32),
        pltpu.SemaphoreType.DMA,
    ],
)(_sc_body)


def _fin_body(x_ref, o_ref):
    p = [jnp.sum(x_ref[4 * k:4 * (k + 1), :]) for k in range(6)]
    n = float(_N)
    rgb_loss = p[0] / n
    mask_loss = (1.0 / _ALPHA) * p[1] / n
    eik_loss = p[2] / n
    contact_loss = p[3] / jnp.maximum(p[4], 1.0)
    contact_reg = p[5] / jnp.maximum((n - p[4]) * 3.0, 1.0)
    o_ref[0, 0] = (_RGB_W * rgb_loss + _MASK_W * mask_loss +
                   _EIK_W * eik_loss + _CSDF_W * contact_loss +
                   _CREG_W * contact_reg)


_finalize = pl.pallas_call(
    _fin_body,
    out_shape=jax.ShapeDtypeStruct((1, 1), jnp.float32),
    out_specs=pl.BlockSpec(memory_space=pltpu.SMEM),
)


@jax.jit
def kernel(rgb_values, rgb_gt, pred_mask, gt_mask, sdf_output, grad_theta,
           sdf_head, sdf_hand, nonrigid_deformation):
    y = jnp.concatenate([rgb_values.T, rgb_gt.T, grad_theta.T,
                         nonrigid_deformation.T], axis=0)
    parts = _sc_partials(
        y, pred_mask.astype(jnp.float32), gt_mask.astype(jnp.float32),
        sdf_output.reshape(-1), sdf_head, sdf_hand)
    total = _finalize(parts.reshape(_NW * 6 * _L // 128, 128))
    return total[0, 0]


# speedup vs baseline: 1.1605x; 1.0552x over previous
import functools

import jax
import jax.numpy as jnp
from jax import lax
from jax.experimental import pallas as pl
from jax.experimental.pallas import tpu as pltpu
from jax.experimental.pallas import tpu_sc as plsc

_N = 65536
_ALPHA = 50.0
_RGB_W = 1.0
_MASK_W = 2.0
_EIK_W = 0.1
_CSDF_W = 1.0
_CREG_W = 1.0

_NC = 2
_NS = 16
_NW = _NC * _NS
_L = 16
_R = _N // _NW
_CH = _R // _L

_AX, _AY, _AZ, _BX, _BY, _BZ, _GX, _GY, _GZ, _NX, _NY, _NZ, \
    _PM, _GM, _SDF, _SH, _SD = range(17)


def _rsqrt(s):
    i = plsc.bitcast(s, jnp.int32)
    i = jnp.int32(0x5F3759DF) - (i >> 1)
    y = plsc.bitcast(i, jnp.float32)
    for _ in range(3):
        y = y * (1.5 - 0.5 * s * y * y)
    return y


def _softplus_neg(a):
    u = jnp.exp(-a)
    s = u / (2.0 + u)
    s2 = s * s
    return 2.0 * s * (1.0 + s2 * (1.0 / 3.0 + s2 * (
        1.0 / 5.0 + s2 * (1.0 / 7.0 + s2 * (1.0 / 9.0)))))


def _sc_body(y, mk, sdf, sh, sd, out, xv, part_v, sem_a):
    wid = lax.axis_index("s") * _NC + lax.axis_index("c")
    base = wid * _R

    copies = [
        pltpu.async_copy(y.at[j, pl.ds(base, _R)],
                         xv.at[pl.ds(j * _R, _R)], sem_a)
        for j in range(12)
    ] + [
        pltpu.async_copy(arr.at[pl.ds(base, _R)],
                         xv.at[pl.ds(j * _R, _R)], sem_a)
        for j, arr in ((_PM, mk), (_SDF, sdf), (_SH, sh), (_SD, sd))
    ]
    for c in copies:
        c.wait()

    zero = jnp.zeros((_L,), jnp.float32)

    def chunk(i, accs):
        a0, a1, a2, a3, a4, a5 = accs
        mk2 = xv[pl.ds(_PM * _R + i * _L, _L)]
        gmv = mk2 - jnp.where(mk2 >= 2.0, 2.0, 0.0)
        m = jnp.where(mk2 >= 3.0, 1.0, 0.0)

        d = (jnp.abs(xv[pl.ds(_AX * _R + i * _L, _L)] - xv[pl.ds(_BX * _R + i * _L, _L)]) +
             jnp.abs(xv[pl.ds(_AY * _R + i * _L, _L)] - xv[pl.ds(_BY * _R + i * _L, _L)]) +
             jnp.abs(xv[pl.ds(_AZ * _R + i * _L, _L)] - xv[pl.ds(_BZ * _R + i * _L, _L)]))
        a0 = a0 + d * m

        z = -_ALPHA * xv[pl.ds(_SDF * _R + i * _L, _L)]
        bce = jnp.maximum(z, 0.0) - z * gmv + _softplus_neg(jnp.abs(z))
        a1 = a1 + bce * (1.0 - m)

        gx = xv[pl.ds(_GX * _R + i * _L, _L)]
        gy = xv[pl.ds(_GY * _R + i * _L, _L)]
        gz = xv[pl.ds(_GZ * _R + i * _L, _L)]
        s = gx * gx + gy * gy + gz * gz
        ns = s * _rsqrt(jnp.maximum(s, 1e-30))
        t = ns - 1.0
        a2 = a2 + t * t

        shv = xv[pl.ds(_SH * _R + i * _L, _L)]
        sdv = xv[pl.ds(_SD * _R + i * _L, _L)]
        cm = jnp.where((shv < 0.0) & (sdv < 0.0), 1.0, 0.0)
        a3 = a3 + jnp.maximum(-shv, 0.0) * cm
        a4 = a4 + cm

        nx = xv[pl.ds(_NX * _R + i * _L, _L)]
        ny = xv[pl.ds(_NY * _R + i * _L, _L)]
        nz = xv[pl.ds(_NZ * _R + i * _L, _L)]
        a5 = a5 + (nx * nx + ny * ny + nz * nz) * (1.0 - cm)

        return (a0, a1, a2, a3, a4, a5)

    accs = lax.fori_loop(0, _CH, chunk, (zero,) * 6)

    for k in range(6):
        part_v[pl.ds(k * _L, _L)] = accs[k]
    outs = [
        pltpu.async_copy(part_v.at[pl.ds(k * _L, _L)],
                         out.at[pl.ds((k * _NW + wid) * _L, _L)], sem_a)
        for k in range(6)
    ]
    for c in outs:
        c.wait()


_sc_partials = functools.partial(
    pl.kernel,
    mesh=plsc.VectorSubcoreMesh(core_axis_name="c", subcore_axis_name="s"),
    out_type=jax.ShapeDtypeStruct((_NW * 6 * _L,), jnp.float32),
    compiler_params=pltpu.CompilerParams(
        needs_layout_passes=False,
        skip_device_barrier=True,
    ),
    scratch_types=[
        pltpu.VMEM((17 * _R,), jnp.float32),
        pltpu.VMEM((6 * _L,), jnp.float32),
        pltpu.SemaphoreType.DMA,
    ],
)(_sc_body)


def _fin_body(x_ref, o_ref):
    p = [jnp.sum(x_ref[4 * k:4 * (k + 1), :]) for k in range(6)]
    n = float(_N)
    rgb_loss = p[0] / n
    mask_loss = (1.0 / _ALPHA) * p[1] / n
    eik_loss = p[2] / n
    contact_loss = p[3] / jnp.maximum(p[4], 1.0)
    contact_reg = p[5] / jnp.maximum((n - p[4]) * 3.0, 1.0)
    o_ref[0, 0] = (_RGB_W * rgb_loss + _MASK_W * mask_loss +
                   _EIK_W * eik_loss + _CSDF_W * contact_loss +
                   _CREG_W * contact_reg)


_finalize = pl.pallas_call(
    _fin_body,
    out_shape=jax.ShapeDtypeStruct((1, 1), jnp.float32),
    out_specs=pl.BlockSpec(memory_space=pltpu.SMEM),
)


@jax.jit
def kernel(rgb_values, rgb_gt, pred_mask, gt_mask, sdf_output, grad_theta,
           sdf_head, sdf_hand, nonrigid_deformation):
    y = jnp.concatenate([rgb_values.T, rgb_gt.T, grad_theta.T,
                         nonrigid_deformation.T], axis=0)
    mk = pred_mask.astype(jnp.float32) * 2.0 + gt_mask.astype(jnp.float32)
    parts = _sc_partials(y, mk, sdf_output.reshape(-1), sdf_head, sdf_hand)
    total = _finalize(parts.reshape(_NW * 6 * _L // 128, 128))
    return total[0, 0]


# speedup vs baseline: 1.2293x; 1.0592x over previous
import functools

import jax
import jax.numpy as jnp
from jax import lax
from jax.experimental import pallas as pl
from jax.experimental.pallas import tpu as pltpu
from jax.experimental.pallas import tpu_sc as plsc

_N = 65536
_ALPHA = 50.0
_RGB_W = 1.0
_MASK_W = 2.0
_EIK_W = 0.1
_CSDF_W = 1.0
_CREG_W = 1.0

_NC = 2
_NS = 16
_NW = _NC * _NS
_L = 16
_R = _N // _NW
_CH = _R // _L

_AX, _AY, _AZ, _BX, _BY, _BZ, _GX, _GY, _GZ, _NX, _NY, _NZ, \
    _PM, _GM, _SDF, _SH, _SD = range(17)


def _rsqrt(s):
    i = plsc.bitcast(s, jnp.int32)
    i = jnp.int32(0x5F3759DF) - (i >> 1)
    y = plsc.bitcast(i, jnp.float32)
    for _ in range(3):
        y = y * (1.5 - 0.5 * s * y * y)
    return y


def _softplus_neg(a):
    u = jnp.exp(-a)
    s = u / (2.0 + u)
    s2 = s * s
    return 2.0 * s * (1.0 + s2 * (1.0 / 3.0 + s2 * (
        1.0 / 5.0 + s2 * (1.0 / 7.0 + s2 * (1.0 / 9.0)))))


def _sc_body(rgb_a, rgb_b, grad, nr, mk, sdf, sh, sd, out,
             cv, sv, part_v, sem_a):
    wid = lax.axis_index("s") * _NC + lax.axis_index("c")
    base = wid * _R

    copies = [
        pltpu.async_copy(arr.at[pl.ds(c, 1), pl.ds(base, _R)],
                         cv.at[pl.ds(a3 * 3 + c, 1), pl.ds(0, _R)],
                         sem_a)
        for a3, arr in enumerate((rgb_a, rgb_b, grad, nr))
        for c in range(3)
    ] + [
        pltpu.async_copy(arr.at[pl.ds(base, _R)],
                         sv.at[pl.ds(j * _R, _R)], sem_a)
        for j, arr in ((0, mk), (1, sdf), (2, sh), (3, sd))
    ]
    for c in copies:
        c.wait()

    iota = lax.iota(jnp.int32, _L)
    rows = [jnp.full((_L,), j, jnp.int32) for j in range(12)]

    def mk_ld(cols):
        def ld(j, i):
            return plsc.load_gather(cv, [rows[j], cols])
        return ld

    zero = jnp.zeros((_L,), jnp.float32)

    def chunk(i, accs):
        a0, a1, a2, a3, a4, a5 = accs
        cols = iota + i * _L
        ld = mk_ld(cols)
        mk2 = sv[pl.ds(0 * _R + i * _L, _L)]
        gmv = mk2 - jnp.where(mk2 >= 2.0, 2.0, 0.0)
        m = jnp.where(mk2 >= 3.0, 1.0, 0.0)

        d = (jnp.abs(ld(0, i) - ld(3, i)) +
             jnp.abs(ld(1, i) - ld(4, i)) +
             jnp.abs(ld(2, i) - ld(5, i)))
        a0 = a0 + d * m

        z = -_ALPHA * sv[pl.ds(1 * _R + i * _L, _L)]
        bce = jnp.maximum(z, 0.0) - z * gmv + _softplus_neg(jnp.abs(z))
        a1 = a1 + bce * (1.0 - m)

        gx = ld(6, i)
        gy = ld(7, i)
        gz = ld(8, i)
        s = gx * gx + gy * gy + gz * gz
        ns = s * _rsqrt(jnp.maximum(s, 1e-30))
        t = ns - 1.0
        a2 = a2 + t * t

        shv = sv[pl.ds(2 * _R + i * _L, _L)]
        sdv = sv[pl.ds(3 * _R + i * _L, _L)]
        cm = jnp.where((shv < 0.0) & (sdv < 0.0), 1.0, 0.0)
        a3 = a3 + jnp.maximum(-shv, 0.0) * cm
        a4 = a4 + cm

        nx = ld(9, i)
        ny = ld(10, i)
        nz = ld(11, i)
        a5 = a5 + (nx * nx + ny * ny + nz * nz) * (1.0 - cm)

        return (a0, a1, a2, a3, a4, a5)

    accs = lax.fori_loop(0, _CH, chunk, (zero,) * 6)

    for k in range(6):
        part_v[pl.ds(k * _L, _L)] = accs[k]
    outs = [
        pltpu.async_copy(part_v.at[pl.ds(k * _L, _L)],
                         out.at[pl.ds((k * _NW + wid) * _L, _L)], sem_a)
        for k in range(6)
    ]
    for c in outs:
        c.wait()


_sc_partials = functools.partial(
    pl.kernel,
    mesh=plsc.VectorSubcoreMesh(core_axis_name="c", subcore_axis_name="s"),
    out_type=jax.ShapeDtypeStruct((_NW * 6 * _L,), jnp.float32),
    compiler_params=pltpu.CompilerParams(
        needs_layout_passes=False,
        skip_device_barrier=True,
    ),
    scratch_types=[
        pltpu.VMEM((16, _R), jnp.float32),
        pltpu.VMEM((4 * _R,), jnp.float32),
        pltpu.VMEM((6 * _L,), jnp.float32),
        pltpu.SemaphoreType.DMA,
    ],
)(_sc_body)


def _fin_body(x_ref, o_ref):
    p = [jnp.sum(x_ref[4 * k:4 * (k + 1), :]) for k in range(6)]
    n = float(_N)
    rgb_loss = p[0] / n
    mask_loss = (1.0 / _ALPHA) * p[1] / n
    eik_loss = p[2] / n
    contact_loss = p[3] / jnp.maximum(p[4], 1.0)
    contact_reg = p[5] / jnp.maximum((n - p[4]) * 3.0, 1.0)
    o_ref[0, 0] = (_RGB_W * rgb_loss + _MASK_W * mask_loss +
                   _EIK_W * eik_loss + _CSDF_W * contact_loss +
                   _CREG_W * contact_reg)


_finalize = pl.pallas_call(
    _fin_body,
    out_shape=jax.ShapeDtypeStruct((1, 1), jnp.float32),
    out_specs=pl.BlockSpec(memory_space=pltpu.SMEM),
)


@jax.jit
def kernel(rgb_values, rgb_gt, pred_mask, gt_mask, sdf_output, grad_theta,
           sdf_head, sdf_hand, nonrigid_deformation):
    mk = pred_mask.astype(jnp.float32) * 2.0 + gt_mask.astype(jnp.float32)
    parts = _sc_partials(rgb_values.T, rgb_gt.T, grad_theta.T,
                         nonrigid_deformation.T, mk,
                         sdf_output.reshape(-1), sdf_head, sdf_hand)
    total = _finalize(parts.reshape(_NW * 6 * _L // 128, 128))
    return total[0, 0]
